# Initial kernel scaffold; baseline (speedup 1.0000x reference)
import jax
import jax.numpy as jnp
from jax.experimental import pallas as pl


def kernel(x, table, W, b):
    raise NotImplementedError("write your pallas kernel here")


# speedup vs baseline: 9.5662x; 9.5662x over previous
import functools

import jax
import jax.numpy as jnp
from jax import lax
from jax.experimental import pallas as pl
from jax.experimental.pallas import tpu as pltpu
from jax.experimental.pallas import tpu_sc as plsc

NC, NS, LANES = 2, 16, 16
NW = NC * NS
B, HIST, D = 16384, 200, 16
VOCAB = 1000000
SPW = B // NW
CS = 16
NCH = SPW // CS
NI = CS * HIST
GW = 128
NG = NI // GW
IPW = SPW * HIST


def _sc_body(x_hbm, table_hbm, wb_hbm, out_hbm,
             idx_a, idx_b, rows_a, rows_b, wb_v, out_v, sem_idx, sem_g):
    cid = lax.axis_index("c")
    sid = lax.axis_index("s")
    wid = sid * NC + cid
    idx0 = wid * IPW

    pltpu.sync_copy(wb_hbm, wb_v)

    idx_bufs = (idx_a, idx_b)
    rows_bufs = (rows_a, rows_b)

    def idx_fire(c, buf):
        pltpu.async_copy(x_hbm.at[pl.ds(idx0 + c * NI, NI)], idx_bufs[buf],
                         sem_idx)

    def idx_wait(c, buf):
        pltpu.make_async_copy(x_hbm.at[pl.ds(idx0 + c * NI, NI)],
                              idx_bufs[buf], sem_idx).wait()

    def gather_fire(buf):
        ib, rb = idx_bufs[buf], rows_bufs[buf]

        def fire(g, carry):
            pltpu.async_copy(table_hbm.at[ib.at[pl.ds(g * GW, GW)]],
                             rb.at[pl.ds(g * GW, GW)], sem_g)
            return carry

        lax.fori_loop(0, NG, fire, 0)

    def gather_drain(buf):
        pltpu.make_async_copy(table_hbm.at[pl.ds(0, NI)], rows_bufs[buf],
                              sem_g).wait()

    lanes = lax.iota(jnp.int32, LANES)

    def accumulate(c, buf):
        rb = rows_bufs[buf]
        wv = wb_v[pl.ds(0, LANES)] * jnp.float32(1.0 / HIST)
        bv = wb_v[pl.ds(LANES, LANES)]

        def sample_body(s, qvec):
            base = s * HIST

            def step(k, accs):
                a0, a1, a2, a3 = accs
                o = base + k * 8
                a0 = a0 + (rb[o] + rb[o + 1])
                a1 = a1 + (rb[o + 2] + rb[o + 3])
                a2 = a2 + (rb[o + 4] + rb[o + 5])
                a3 = a3 + (rb[o + 6] + rb[o + 7])
                return (a0, a1, a2, a3)

            z16 = jnp.zeros((LANES,), jnp.float32)
            a0, a1, a2, a3 = lax.fori_loop(0, HIST // 8, step,
                                           (z16, z16, z16, z16))
            acc = (a0 + a1) + (a2 + a3)
            q = jnp.sum(acc * wv)
            return jnp.where(lanes == s, q, qvec)

        qvec = lax.fori_loop(0, CS, sample_body,
                             jnp.zeros((LANES,), jnp.float32))
        z = qvec + bv
        e = jnp.exp(-jnp.abs(z))
        sp = jnp.float32(1.0) / (jnp.float32(1.0) + e)
        res = jnp.where(z >= 0, sp, jnp.float32(1.0) - sp)
        yi = (res * jnp.float32(1e4) + jnp.float32(0.5)).astype(jnp.int32)
        out_v[pl.ds(c * CS, CS)] = yi.astype(jnp.float32) / jnp.float32(1e4)

    idx_fire(0, 0)
    idx_fire(1, 1)
    idx_wait(0, 0)
    gather_fire(0)

    def subchunk(c, buf):
        gather_drain(buf)
        idx_wait(c + 1, 1 - buf)
        gather_fire(1 - buf)
        idx_fire(c + 2, buf)
        accumulate(c, buf)

    def outer(i, carry):
        c = 2 * i
        subchunk(c, 0)
        subchunk(c + 1, 1)
        return carry

    lax.fori_loop(0, (NCH - 2) // 2, outer, 0)

    c_last = jnp.int32(NCH - 2)
    gather_drain(0)
    idx_wait(jnp.int32(NCH - 1), 1)
    gather_fire(1)
    accumulate(c_last, 0)
    gather_drain(1)
    accumulate(c_last + 1, 1)

    pltpu.sync_copy(out_v, out_hbm.at[pl.ds(wid * SPW, SPW)])


@functools.partial(jax.jit, static_argnames=())
def kernel(x, table, W, b):
    assert x.shape == (B, HIST) and table.shape[1] == D
    x_flat = x.reshape(-1).astype(jnp.int32)
    wb = jnp.concatenate([W.reshape(D).astype(jnp.float32),
                          jnp.broadcast_to(b.reshape(1).astype(jnp.float32),
                                           (LANES,))])
    mesh = plsc.VectorSubcoreMesh(core_axis_name="c", subcore_axis_name="s",
                                  num_cores=NC, num_subcores=NS)
    kfn = pl.kernel(
        _sc_body,
        out_type=jax.ShapeDtypeStruct((B,), jnp.float32),
        mesh=mesh,
        compiler_params=pltpu.CompilerParams(needs_layout_passes=False,
                                             use_tc_tiling_on_sc=False),
        scratch_types=[
            pltpu.VMEM((NI,), jnp.int32),
            pltpu.VMEM((NI,), jnp.int32),
            pltpu.VMEM((NI, D), jnp.float32),
            pltpu.VMEM((NI, D), jnp.float32),
            pltpu.VMEM((2 * LANES,), jnp.float32),
            pltpu.VMEM((SPW,), jnp.float32),
            pltpu.SemaphoreType.DMA,
            pltpu.SemaphoreType.DMA,
        ],
    )
    out = kfn(x_flat, table, wb)
    return out.reshape(B, 1)
